# Initial kernel scaffold; baseline (speedup 1.0000x reference)
import jax
import jax.numpy as jnp
from jax.experimental import pallas as pl


def kernel(tokens, embedding_table):
    raise NotImplementedError("write your pallas kernel here")


# speedup vs baseline: 3.8291x; 3.8291x over previous
import functools
import math

import jax
import jax.numpy as jnp
from jax import lax
from jax.experimental import pallas as pl
from jax.experimental.pallas import tpu as pltpu
from jax.experimental.pallas import tpu_sc as plsc

VOCAB = 100000
D = 64
SCALE = math.sqrt(D)

NC = 2
NS = 16
NW = NC * NS

B_TOK = 4096 * 200
PER_W = B_TOK // NW
KROWS = 8
CHUNK = KROWS * 128
NCHUNK = PER_W // CHUNK
IDX_ROWS_PER_W = PER_W // 128


def _scale_body(t_ref, o_ref):
    o_ref[...] = t_ref[...] * SCALE


def _scale_table(table):
    rows = table.shape[0]
    blk = 4000
    grid = rows // blk
    return pl.pallas_call(
        _scale_body,
        grid=(grid,),
        in_specs=[pl.BlockSpec((blk, D), lambda i: (i, 0))],
        out_specs=pl.BlockSpec((blk, D), lambda i: (i, 0)),
        out_shape=jax.ShapeDtypeStruct((rows, D), jnp.float32),
    )(table)


def _gather_body(tok_hbm, table_hbm, out_hbm, idx_v, rows_v, gsem, ssem):
    wid = lax.axis_index("s") * NC + lax.axis_index("c")
    out_base = wid * PER_W

    def chunk_step(g, carry):
        pltpu.sync_copy(tok_hbm.at[wid, pl.ds(g * KROWS, KROWS)], idx_v)
        descs = [
            pltpu.make_async_copy(
                table_hbm.at[idx_v.at[j]],
                rows_v.at[pl.ds(j * 128, 128)],
                gsem,
            )
            for j in range(KROWS)
        ]
        for d in descs:
            d.start()
        for d in descs:
            d.wait()
        out = pltpu.make_async_copy(
            rows_v, out_hbm.at[pl.ds(out_base + g * CHUNK, CHUNK)], ssem
        )
        out.start()
        out.wait()
        return carry

    lax.fori_loop(0, NCHUNK, chunk_step, 0)


def _sc_gather(tok3d, scaled_table):
    kern = functools.partial(
        pl.kernel,
        mesh=plsc.VectorSubcoreMesh(core_axis_name="c", subcore_axis_name="s"),
        out_type=jax.ShapeDtypeStruct((B_TOK, D), jnp.float32),
        scratch_types=[
            pltpu.VMEM((KROWS, 128), jnp.int32),
            pltpu.VMEM((CHUNK, D), jnp.float32),
            pltpu.SemaphoreType.DMA,
            pltpu.SemaphoreType.DMA,
        ],
        compiler_params=pltpu.CompilerParams(use_tc_tiling_on_sc=False),
    )(_gather_body)
    return kern(tok3d, scaled_table)


def kernel(tokens, embedding_table):
    tok3d = tokens.astype(jnp.int32).reshape(NW, IDX_ROWS_PER_W, 128)
    scaled = _scale_table(embedding_table)
    flat = _sc_gather(tok3d, scaled)
    return flat.reshape(tokens.shape[0], tokens.shape[1], D)


# speedup vs baseline: 3.9383x; 1.0285x over previous
import functools
import math

import jax
import jax.numpy as jnp
from jax import lax
from jax.experimental import pallas as pl
from jax.experimental.pallas import tpu as pltpu
from jax.experimental.pallas import tpu_sc as plsc

VOCAB = 100000
D = 64
SCALE = math.sqrt(D)

NC = 2
NS = 16
NW = NC * NS

B_TOK = 4096 * 200
PER_W = B_TOK // NW
IDX_ROWS_PER_W = PER_W // 128
HCROWS = 4
HC = HCROWS * 128
NHC = PER_W // HC


def _scale_body(t_ref, o_ref):
    o_ref[...] = t_ref[...] * SCALE


def _scale_table(table):
    rows = table.shape[0]
    blk = 4000
    grid = rows // blk
    return pl.pallas_call(
        _scale_body,
        grid=(grid,),
        in_specs=[pl.BlockSpec((blk, D), lambda i: (i, 0))],
        out_specs=pl.BlockSpec((blk, D), lambda i: (i, 0)),
        out_shape=jax.ShapeDtypeStruct((rows, D), jnp.float32),
    )(table)


def _gather_body(tok_hbm, table_hbm, out_hbm, idx_v, r0, r1, g0, g1, s0, s1):
    wid = lax.axis_index("s") * NC + lax.axis_index("c")
    out_base = wid * PER_W

    def gather_descs(h, rbuf, gsem):
        return [
            pltpu.make_async_copy(
                table_hbm.at[idx_v.at[h * HCROWS + k]],
                rbuf.at[pl.ds(k * 128, 128)],
                gsem,
            )
            for k in range(HCROWS)
        ]

    def start_gathers(h, rbuf, gsem):
        for d in gather_descs(h, rbuf, gsem):
            d.start()

    def wait_gathers(h, rbuf, gsem):
        for d in gather_descs(h, rbuf, gsem):
            d.wait()

    def scatter_desc(h, rbuf, ssem):
        return pltpu.make_async_copy(
            rbuf, out_hbm.at[pl.ds(out_base + h * HC, HC)], ssem
        )

    pltpu.sync_copy(tok_hbm.at[wid], idx_v)

    start_gathers(0, r0, g0)
    start_gathers(1, r1, g1)
    wait_gathers(0, r0, g0)
    scatter_desc(0, r0, s0).start()

    def step(p, carry):
        h = 2 * p
        scatter_desc(h, r0, s0).wait()
        start_gathers(h, r0, g0)
        wait_gathers(h - 1, r1, g1)
        scatter_desc(h - 1, r1, s1).start()
        scatter_desc(h - 1, r1, s1).wait()
        start_gathers(h + 1, r1, g1)
        wait_gathers(h, r0, g0)
        scatter_desc(h, r0, s0).start()
        return carry

    lax.fori_loop(1, NHC // 2, step, 0)

    wait_gathers(NHC - 1, r1, g1)
    scatter_desc(NHC - 1, r1, s1).start()
    scatter_desc(NHC - 2, r0, s0).wait()
    scatter_desc(NHC - 1, r1, s1).wait()


def _sc_gather(tok3d, scaled_table):
    kern = functools.partial(
        pl.kernel,
        mesh=plsc.VectorSubcoreMesh(core_axis_name="c", subcore_axis_name="s"),
        out_type=jax.ShapeDtypeStruct((B_TOK, D), jnp.float32),
        scratch_types=[
            pltpu.VMEM((IDX_ROWS_PER_W, 128), jnp.int32),
            pltpu.VMEM((HC, D), jnp.float32),
            pltpu.VMEM((HC, D), jnp.float32),
            pltpu.SemaphoreType.DMA,
            pltpu.SemaphoreType.DMA,
            pltpu.SemaphoreType.DMA,
            pltpu.SemaphoreType.DMA,
        ],
        compiler_params=pltpu.CompilerParams(use_tc_tiling_on_sc=False),
    )(_gather_body)
    return kern(tok3d, scaled_table)


def kernel(tokens, embedding_table):
    tok3d = tokens.astype(jnp.int32).reshape(NW, IDX_ROWS_PER_W, 128)
    scaled = _scale_table(embedding_table)
    flat = _sc_gather(tok3d, scaled)
    return flat.reshape(tokens.shape[0], tokens.shape[1], D)


# speedup vs baseline: 6.1152x; 1.5528x over previous
import functools
import math

import jax
import jax.numpy as jnp
from jax import lax
from jax.experimental import pallas as pl
from jax.experimental.pallas import tpu as pltpu
from jax.experimental.pallas import tpu_sc as plsc

VOCAB = 100000
D = 64
SCALE = math.sqrt(D)

NC = 2
NS = 16
NW = NC * NS

B_SEQ = 4096
T_SEQ = 200
B_TOK = B_SEQ * T_SEQ
PER_W = B_TOK // NW
IDX_ROWS_PER_W = PER_W // 128
HCROWS = 4
HC = HCROWS * 128
NHC = PER_W // HC


def _gather_body(tok_hbm, dst_hbm, table_hbm, out_hbm,
                 idx_v, dst_v, r0, r1, g0, g1, s0, s1):
    wid = lax.axis_index("s") * NC + lax.axis_index("c")

    def gather_descs(h, rbuf, gsem):
        return [
            pltpu.make_async_copy(
                table_hbm.at[idx_v.at[h * HCROWS + k]],
                rbuf.at[pl.ds(k * 128, 128)],
                gsem,
            )
            for k in range(HCROWS)
        ]

    def scatter_descs(h, rbuf, ssem):
        return [
            pltpu.make_async_copy(
                rbuf.at[pl.ds(k * 128, 128)],
                out_hbm.at[dst_v.at[h * HCROWS + k]],
                ssem,
            )
            for k in range(HCROWS)
        ]

    def start(ds):
        for d in ds:
            d.start()

    def wait(ds):
        for d in ds:
            d.wait()

    pltpu.sync_copy(tok_hbm.at[wid], idx_v)
    pltpu.sync_copy(dst_hbm.at[wid], dst_v)

    start(gather_descs(0, r0, g0))
    start(gather_descs(1, r1, g1))
    wait(gather_descs(0, r0, g0))
    start(scatter_descs(0, r0, s0))

    def step(p, carry):
        h = 2 * p
        wait(scatter_descs(h, r0, s0))
        start(gather_descs(h, r0, g0))
        wait(gather_descs(h - 1, r1, g1))
        start(scatter_descs(h - 1, r1, s1))
        wait(scatter_descs(h - 1, r1, s1))
        start(gather_descs(h + 1, r1, g1))
        wait(gather_descs(h, r0, g0))
        start(scatter_descs(h, r0, s0))
        return carry

    lax.fori_loop(1, NHC // 2, step, 0)

    wait(gather_descs(NHC - 1, r1, g1))
    start(scatter_descs(NHC - 1, r1, s1))
    wait(scatter_descs(NHC - 2, r0, s0))
    wait(scatter_descs(NHC - 1, r1, s1))


def _sc_gather(tok3d, dst3d, table):
    kern = functools.partial(
        pl.kernel,
        mesh=plsc.VectorSubcoreMesh(core_axis_name="c", subcore_axis_name="s"),
        out_type=jax.ShapeDtypeStruct((B_TOK, D), jnp.float32),
        scratch_types=[
            pltpu.VMEM((IDX_ROWS_PER_W, 128), jnp.int32),
            pltpu.VMEM((IDX_ROWS_PER_W, 128), jnp.int32),
            pltpu.VMEM((HC, D), jnp.float32),
            pltpu.VMEM((HC, D), jnp.float32),
            pltpu.SemaphoreType.DMA,
            pltpu.SemaphoreType.DMA,
            pltpu.SemaphoreType.DMA,
            pltpu.SemaphoreType.DMA,
        ],
        compiler_params=pltpu.CompilerParams(use_tc_tiling_on_sc=False),
    )(_gather_body)
    return kern(tok3d, dst3d, table)


def _tr_body(g_ref, o_ref):
    for s in range(4):
        xs = g_ref[pl.ds(512 * s, 512), :]
        xl = xs[:, 0:64]
        xr = xs[:, 64:128]
        o_ref[0, :, pl.ds(512 * s, 512)] = xl.T * SCALE
        o_ref[0, :, pl.ds(2048 + 512 * s, 512)] = xr.T * SCALE


def _tc_transpose(g2):
    return pl.pallas_call(
        _tr_body,
        grid=(T_SEQ,),
        in_specs=[pl.BlockSpec((B_SEQ // 2, 128), lambda i: (i, 0))],
        out_specs=pl.BlockSpec((1, D, B_SEQ), lambda i: (i, 0, 0)),
        out_shape=jax.ShapeDtypeStruct((T_SEQ, D, B_SEQ), jnp.float32),
    )(g2)


def kernel(tokens, embedding_table):
    tok3d = tokens.astype(jnp.int32).reshape(NW, IDX_ROWS_PER_W, 128)
    m = jnp.arange(B_TOK, dtype=jnp.int32)
    b = m // T_SEQ
    t = m % T_SEQ
    dst = t * B_SEQ + (b % (B_SEQ // 2)) * 2 + b // (B_SEQ // 2)
    dst3d = dst.reshape(NW, IDX_ROWS_PER_W, 128)
    gt = _sc_gather(tok3d, dst3d, embedding_table)
    g2 = gt.reshape(B_TOK // 2, 128)
    outp = _tc_transpose(g2)
    return jnp.transpose(outp, (2, 0, 1))
